# speedup vs baseline: 1.2733x; 1.2733x over previous
import functools
import math

import jax
import jax.numpy as jnp
from jax import lax
from jax.experimental import pallas as pl
from jax.experimental.pallas import tpu as pltpu
from jax.experimental.pallas import tpu_sc as plsc

D_MODEL = 768
SCALE = math.sqrt(D_MODEL)
LANES = 16
VECS_PER_ROW = D_MODEL // LANES

NUM_WORKERS = 32
CHUNK = 128


def _make_gather(B):
    b_per_w = B // NUM_WORKERS
    n_chunks = b_per_w // CHUNK
    mesh = plsc.VectorSubcoreMesh(core_axis_name="c", subcore_axis_name="s")

    @functools.partial(
        pl.kernel,
        mesh=mesh,
        out_type=jax.ShapeDtypeStruct((B, D_MODEL), jnp.float32),
        scratch_types=[
            pltpu.VMEM((b_per_w,), jnp.int32),
            pltpu.VMEM((CHUNK, D_MODEL), jnp.float32),
            pltpu.SemaphoreType.DMA,
        ],
    )
    def gather_kernel(idx_hbm, table_hbm, out_hbm, idx_v, rows_v, sem):
        wid = lax.axis_index("s") * 2 + lax.axis_index("c")
        base = wid * b_per_w
        pltpu.sync_copy(idx_hbm.at[pl.ds(base, b_per_w)], idx_v)

        @pl.loop(0, n_chunks)
        def _chunk(c):
            row0 = c * CHUNK
            pltpu.async_copy(
                table_hbm.at[idx_v.at[pl.ds(row0, CHUNK)]], rows_v, sem
            ).wait()

            @pl.loop(0, CHUNK)
            def _scale(r):
                for j in range(VECS_PER_ROW):
                    sl = pl.ds(j * LANES, LANES)
                    rows_v[r, sl] = rows_v[r, sl] * SCALE

            pltpu.sync_copy(rows_v, out_hbm.at[pl.ds(base + row0, CHUNK)])

    return gather_kernel


def kernel(x, table):
    b, s = x.shape
    idx = x.reshape(-1).astype(jnp.int32)
    out = _make_gather(b * s)(idx, table)
    return out.reshape(b, s, D_MODEL)


# speedup vs baseline: 1.5873x; 1.2466x over previous
import functools
import math

import jax
import jax.numpy as jnp
from jax import lax
from jax.experimental import pallas as pl
from jax.experimental.pallas import tpu as pltpu
from jax.experimental.pallas import tpu_sc as plsc

D_MODEL = 768
SCALE = math.sqrt(D_MODEL)
LANES = 16
VECS_PER_ROW = D_MODEL // LANES

NUM_WORKERS = 32
CHUNK = 64


def _make_gather(B):
    b_per_w = B // NUM_WORKERS
    n_chunks = b_per_w // CHUNK
    mesh = plsc.VectorSubcoreMesh(core_axis_name="c", subcore_axis_name="s")

    @functools.partial(
        pl.kernel,
        mesh=mesh,
        out_type=jax.ShapeDtypeStruct((B, D_MODEL), jnp.float32),
        scratch_types=[
            pltpu.VMEM((b_per_w,), jnp.int32),
            pltpu.VMEM((CHUNK, D_MODEL), jnp.float32),
            pltpu.VMEM((CHUNK, D_MODEL), jnp.float32),
            pltpu.SemaphoreType.DMA,
            pltpu.SemaphoreType.DMA,
        ],
    )
    def gather_kernel(idx_hbm, table_hbm, out_hbm, idx_v, rows0, rows1,
                      gsem, ssem):
        wid = lax.axis_index("s") * 2 + lax.axis_index("c")
        base = wid * b_per_w
        pltpu.sync_copy(idx_hbm.at[pl.ds(base, b_per_w)], idx_v)

        bufs = (rows0, rows1)

        def start_gather(c, buf):
            return pltpu.async_copy(
                table_hbm.at[idx_v.at[pl.ds(c * CHUNK, CHUNK)]], buf, gsem)

        gather_cp = [None, None]
        store_cp = [None, None]
        gather_cp[0] = start_gather(0, bufs[0])
        for c in range(n_chunks):
            b = c % 2
            if c + 1 < n_chunks:
                nb = (c + 1) % 2
                if store_cp[nb] is not None:
                    store_cp[nb].wait()
                gather_cp[nb] = start_gather(c + 1, bufs[nb])
            gather_cp[b].wait()
            buf = bufs[b]

            @pl.loop(0, CHUNK)
            def _scale(r):
                for j in range(VECS_PER_ROW):
                    sl = pl.ds(j * LANES, LANES)
                    buf[r, sl] = buf[r, sl] * SCALE

            store_cp[b] = pltpu.async_copy(
                buf, out_hbm.at[pl.ds(base + c * CHUNK, CHUNK)], ssem)
        store_cp[(n_chunks - 2) % 2].wait()
        store_cp[(n_chunks - 1) % 2].wait()

    return gather_kernel


def kernel(x, table):
    b, s = x.shape
    idx = x.reshape(-1).astype(jnp.int32)
    out = _make_gather(b * s)(idx, table)
    return out.reshape(b, s, D_MODEL)
